# baseline (device time: 122898 ns/iter reference)
import jax
import jax.numpy as jnp
from jax import lax
from jax.experimental import pallas as pl
from jax.experimental.pallas import tpu as pltpu

ROWS = 64
CW = 512


def kernel(x, W):
    m, k = x.shape
    _, n_local = W.shape
    n_global = 2 * n_local
    n_rc = m // ROWS
    n_wc = n_local // CW

    def body(x_ref, w_ref, out_ref, xb_ref, wb_ref, wstage_ref,
             loc_ref, rem_ref, wsems, tsend, trecv, rsend, rrecv):
        my_x = lax.axis_index("x")
        my_y = lax.axis_index("y")
        my_z = lax.axis_index("z")
        peer = (1 - my_x, my_y, my_z)

        barrier = pltpu.get_barrier_semaphore()
        pl.semaphore_signal(barrier, inc=1, device_id=peer,
                            device_id_type=pltpu.DeviceIdType.MESH)

        xb_ref[:, :] = x_ref[:, :].astype(jnp.bfloat16)

        wdmas = [
            pltpu.make_async_copy(
                w_ref.at[:, pl.ds(c * CW, CW)],
                wstage_ref.at[c % 2],
                wsems.at[c % 2],
            )
            for c in range(n_wc)
        ]
        wdmas[0].start()
        wdmas[1].start()

        r0 = pl.ds(0, ROWS)
        tiles = []
        for c in range(n_wc):
            cs = pl.ds(c * CW, CW)
            wdmas[c].wait()
            wchunk = wstage_ref[c % 2].astype(jnp.bfloat16)
            wb_ref[:, cs] = wchunk
            if c + 2 < n_wc:
                wdmas[c + 2].start()
            t = jax.lax.dot(xb_ref[r0, :], wchunk,
                            preferred_element_type=jnp.float32)
            loc_ref[r0, cs] = t.astype(jnp.bfloat16)
            if c == 0:
                pl.semaphore_wait(barrier, 1)
            rdma = pltpu.make_async_remote_copy(
                src_ref=loc_ref.at[r0, cs],
                dst_ref=rem_ref.at[r0, cs],
                send_sem=tsend.at[c],
                recv_sem=trecv.at[c],
                device_id=peer,
                device_id_type=pltpu.DeviceIdType.MESH,
            )
            rdma.start()
            tiles.append(rdma)

        rowsends = []
        for r in range(1, n_rc):
            rs = pl.ds(r * ROWS, ROWS)
            logits = jax.lax.dot(xb_ref[rs, :], wb_ref[:, :],
                                 preferred_element_type=jnp.float32)
            loc_ref[rs, :] = logits.astype(jnp.bfloat16)
            rdma = pltpu.make_async_remote_copy(
                src_ref=loc_ref.at[rs],
                dst_ref=rem_ref.at[rs],
                send_sem=rsend.at[r - 1],
                recv_sem=rrecv.at[r - 1],
                device_id=peer,
                device_id_type=pltpu.DeviceIdType.MESH,
            )
            rdma.start()
            rowsends.append(rdma)

        for r in range(n_rc):
            rs = pl.ds(r * ROWS, ROWS)
            el = jnp.exp(loc_ref[rs, :])
            sl = jnp.sum(el.astype(jnp.float32), axis=-1, keepdims=True)
            if r == 0:
                for t_ in tiles:
                    t_.wait_recv()
            else:
                rowsends[r - 1].wait_recv()
            er = jnp.exp(rem_ref[rs, :])
            sr = jnp.sum(er.astype(jnp.float32), axis=-1, keepdims=True)
            inv = (1.0 / (sl + sr)).astype(jnp.bfloat16)
            elf = el * inv
            erf = er * inv

            @pl.when(my_x == 0)
            def _():
                out_ref[rs, :n_local] = elf
                out_ref[rs, n_local:] = erf

            @pl.when(my_x != 0)
            def _():
                out_ref[rs, :n_local] = erf
                out_ref[rs, n_local:] = elf

        for t_ in tiles:
            t_.wait_send()
        for r_ in rowsends:
            r_.wait_send()

    return pl.pallas_call(
        body,
        out_shape=jax.ShapeDtypeStruct((m, n_global), jnp.bfloat16),
        in_specs=[
            pl.BlockSpec(memory_space=pltpu.VMEM),
            pl.BlockSpec(memory_space=pl.ANY),
        ],
        out_specs=pl.BlockSpec(memory_space=pltpu.VMEM),
        scratch_shapes=[
            pltpu.VMEM((m, k), jnp.bfloat16),
            pltpu.VMEM((k, n_local), jnp.bfloat16),
            pltpu.VMEM((2, k, CW), jnp.float32),
            pltpu.VMEM((m, n_local), jnp.bfloat16),
            pltpu.VMEM((m, n_local), jnp.bfloat16),
            pltpu.SemaphoreType.DMA((2,)),
            pltpu.SemaphoreType.DMA((n_wc,)),
            pltpu.SemaphoreType.DMA((n_wc,)),
            pltpu.SemaphoreType.DMA((n_rc - 1,)),
            pltpu.SemaphoreType.DMA((n_rc - 1,)),
        ],
        compiler_params=pltpu.CompilerParams(
            collective_id=0,
            vmem_limit_bytes=63 * 1024 * 1024,
        ),
    )(x, W)


# device time: 121360 ns/iter; 1.0127x vs baseline; 1.0127x over previous
import jax
import jax.numpy as jnp
from jax import lax
from jax.experimental import pallas as pl
from jax.experimental.pallas import tpu as pltpu

ROWS = 64
CW = 1024


def kernel(x, W):
    m, k = x.shape
    _, n_local = W.shape
    n_global = 2 * n_local
    n_rc = m // ROWS
    n_wc = n_local // CW

    def body(x_ref, w_ref, dummy_ref, out_ref, xb_ref, wb_ref, wstage_ref,
             loc_ref, rem_ref, obuf_ref,
             wsems, tsend, trecv, rsend, rrecv, copy_sems):
        del dummy_ref
        my_x = lax.axis_index("x")
        my_y = lax.axis_index("y")
        my_z = lax.axis_index("z")
        peer = (1 - my_x, my_y, my_z)

        barrier = pltpu.get_barrier_semaphore()
        pl.semaphore_signal(barrier, inc=1, device_id=peer,
                            device_id_type=pltpu.DeviceIdType.MESH)

        xb_ref[:, :] = x_ref[:, :].astype(jnp.bfloat16)

        wdmas = [
            pltpu.make_async_copy(
                w_ref.at[:, pl.ds(c * CW, CW)],
                wstage_ref.at[c % 2],
                wsems.at[c % 2],
            )
            for c in range(n_wc)
        ]
        wdmas[0].start()
        wdmas[1].start()

        r0 = pl.ds(0, ROWS)
        tiles = []
        for c in range(n_wc):
            cs = pl.ds(c * CW, CW)
            wdmas[c].wait()
            wchunk = wstage_ref[c % 2].astype(jnp.bfloat16)
            wb_ref[:, cs] = wchunk
            if c + 2 < n_wc:
                wdmas[c + 2].start()
            t = jax.lax.dot(xb_ref[r0, :], wchunk,
                            preferred_element_type=jnp.float32)
            loc_ref[r0, cs] = t.astype(jnp.bfloat16)
            if c == 0:
                pl.semaphore_wait(barrier, 1)
            rdma = pltpu.make_async_remote_copy(
                src_ref=loc_ref.at[r0, cs],
                dst_ref=rem_ref.at[r0, cs],
                send_sem=tsend.at[c],
                recv_sem=trecv.at[c],
                device_id=peer,
                device_id_type=pltpu.DeviceIdType.MESH,
            )
            rdma.start()
            tiles.append(rdma)

        rowsends = []
        for r in range(1, n_rc):
            rs = pl.ds(r * ROWS, ROWS)
            logits = jax.lax.dot(xb_ref[rs, :], wb_ref[:, :],
                                 preferred_element_type=jnp.float32)
            loc_ref[rs, :] = logits.astype(jnp.bfloat16)
            rdma = pltpu.make_async_remote_copy(
                src_ref=loc_ref.at[rs],
                dst_ref=rem_ref.at[rs],
                send_sem=rsend.at[r - 1],
                recv_sem=rrecv.at[r - 1],
                device_id=peer,
                device_id_type=pltpu.DeviceIdType.MESH,
            )
            rdma.start()
            rowsends.append(rdma)

        copies = []
        for r in range(n_rc):
            rs = pl.ds(r * ROWS, ROWS)
            slot = r % 2
            el = jnp.exp(loc_ref[rs, :])
            sl = jnp.sum(el.astype(jnp.float32), axis=-1, keepdims=True)
            if r == 0:
                for t_ in tiles:
                    t_.wait_recv()
            else:
                rowsends[r - 1].wait_recv()
            if r >= 2:
                copies[r - 2].wait()
            er = jnp.exp(rem_ref[rs, :])
            sr = jnp.sum(er.astype(jnp.float32), axis=-1, keepdims=True)
            inv = (1.0 / (sl + sr)).astype(jnp.bfloat16)
            elf = el * inv
            erf = er * inv

            @pl.when(my_x == 0)
            def _():
                obuf_ref[slot, :, :n_local] = elf
                obuf_ref[slot, :, n_local:] = erf

            @pl.when(my_x != 0)
            def _():
                obuf_ref[slot, :, :n_local] = erf
                obuf_ref[slot, :, n_local:] = elf

            copy = pltpu.make_async_copy(
                obuf_ref.at[slot], out_ref.at[rs], copy_sems.at[slot]
            )
            copy.start()
            copies.append(copy)

        copies[-2].wait()
        copies[-1].wait()
        for t_ in tiles:
            t_.wait_send()
        for r_ in rowsends:
            r_.wait_send()

    return pl.pallas_call(
        body,
        out_shape=jax.ShapeDtypeStruct((m, n_global), jnp.bfloat16),
        in_specs=[
            pl.BlockSpec(memory_space=pltpu.VMEM),
            pl.BlockSpec(memory_space=pl.ANY),
            pl.BlockSpec(memory_space=pl.ANY),
        ],
        out_specs=pl.BlockSpec(memory_space=pl.ANY),
        input_output_aliases={2: 0},
        scratch_shapes=[
            pltpu.VMEM((m, k), jnp.bfloat16),
            pltpu.VMEM((k, n_local), jnp.bfloat16),
            pltpu.VMEM((2, k, CW), jnp.float32),
            pltpu.VMEM((m, n_local), jnp.bfloat16),
            pltpu.VMEM((m, n_local), jnp.bfloat16),
            pltpu.VMEM((2, ROWS, n_global), jnp.bfloat16),
            pltpu.SemaphoreType.DMA((2,)),
            pltpu.SemaphoreType.DMA((n_wc,)),
            pltpu.SemaphoreType.DMA((n_wc,)),
            pltpu.SemaphoreType.DMA((n_rc - 1,)),
            pltpu.SemaphoreType.DMA((n_rc - 1,)),
            pltpu.SemaphoreType.DMA((2,)),
        ],
        compiler_params=pltpu.CompilerParams(
            collective_id=0,
            vmem_limit_bytes=63 * 1024 * 1024,
        ),
    )(x, W, jnp.zeros((m, n_global), jnp.bfloat16))


# device time: 114849 ns/iter; 1.0701x vs baseline; 1.0567x over previous
import jax
import jax.numpy as jnp
from jax import lax
from jax.experimental import pallas as pl
from jax.experimental.pallas import tpu as pltpu

ROWS = 64
CW = 1024


def kernel(x, W):
    m, k = x.shape
    _, n_local = W.shape
    n_global = 2 * n_local
    n_rc = m // ROWS
    n_wc = n_local // CW

    def body(x_ref, w_ref, out_ref, xb_ref, wb_ref, wstage_ref,
             loc_ref, rem_ref, obuf_ref,
             wsems, tsend, trecv, rsend, rrecv, copy_sems):
        my_x = lax.axis_index("x")
        my_y = lax.axis_index("y")
        my_z = lax.axis_index("z")
        peer = (1 - my_x, my_y, my_z)

        barrier = pltpu.get_barrier_semaphore()
        pl.semaphore_signal(barrier, inc=1, device_id=peer,
                            device_id_type=pltpu.DeviceIdType.MESH)

        xb_ref[:, :] = x_ref[:, :].astype(jnp.bfloat16)

        wdmas = [
            pltpu.make_async_copy(
                w_ref.at[:, pl.ds(c * CW, CW)],
                wstage_ref.at[c % 2],
                wsems.at[c % 2],
            )
            for c in range(n_wc)
        ]
        wdmas[0].start()
        wdmas[1].start()

        r0 = pl.ds(0, ROWS)
        tiles = []
        for c in range(n_wc):
            cs = pl.ds(c * CW, CW)
            wdmas[c].wait()
            wchunk = wstage_ref[c % 2].astype(jnp.bfloat16)
            wb_ref[:, cs] = wchunk
            if c + 2 < n_wc:
                wdmas[c + 2].start()
            t = jax.lax.dot(xb_ref[r0, :], wchunk,
                            preferred_element_type=jnp.float32)
            loc_ref[r0, cs] = t.astype(jnp.bfloat16)
            if c == 0:
                pl.semaphore_wait(barrier, 1)
            rdma = pltpu.make_async_remote_copy(
                src_ref=loc_ref.at[r0, cs],
                dst_ref=rem_ref.at[r0, cs],
                send_sem=tsend.at[c],
                recv_sem=trecv.at[c],
                device_id=peer,
                device_id_type=pltpu.DeviceIdType.MESH,
            )
            rdma.start()
            tiles.append(rdma)

        rowsends = []
        for r in range(1, n_rc):
            rs = pl.ds(r * ROWS, ROWS)
            logits = jax.lax.dot(xb_ref[rs, :], wb_ref[:, :],
                                 preferred_element_type=jnp.float32)
            loc_ref[rs, :] = logits.astype(jnp.bfloat16)
            rdma = pltpu.make_async_remote_copy(
                src_ref=loc_ref.at[rs],
                dst_ref=rem_ref.at[rs],
                send_sem=rsend.at[r - 1],
                recv_sem=rrecv.at[r - 1],
                device_id=peer,
                device_id_type=pltpu.DeviceIdType.MESH,
            )
            rdma.start()
            rowsends.append(rdma)

        copies = []
        for r in range(n_rc):
            rs = pl.ds(r * ROWS, ROWS)
            slot = r % 2
            el = jnp.exp(loc_ref[rs, :])
            sl = jnp.sum(el.astype(jnp.float32), axis=-1, keepdims=True)
            if r == 0:
                for t_ in tiles:
                    t_.wait_recv()
            else:
                rowsends[r - 1].wait_recv()
            if r >= 2:
                copies[r - 2].wait()
            er = jnp.exp(rem_ref[rs, :])
            sr = jnp.sum(er.astype(jnp.float32), axis=-1, keepdims=True)
            inv = (1.0 / (sl + sr)).astype(jnp.bfloat16)
            elf = el * inv
            erf = er * inv

            @pl.when(my_x == 0)
            def _():
                obuf_ref[slot, :, :n_local] = elf
                obuf_ref[slot, :, n_local:] = erf

            @pl.when(my_x != 0)
            def _():
                obuf_ref[slot, :, :n_local] = erf
                obuf_ref[slot, :, n_local:] = elf

            copy = pltpu.make_async_copy(
                obuf_ref.at[slot], out_ref.at[rs], copy_sems.at[slot]
            )
            copy.start()
            copies.append(copy)

        copies[-2].wait()
        copies[-1].wait()
        for t_ in tiles:
            t_.wait_send()
        for r_ in rowsends:
            r_.wait_send()

    return pl.pallas_call(
        body,
        out_shape=jax.ShapeDtypeStruct((m, n_global), jnp.bfloat16),
        in_specs=[
            pl.BlockSpec(memory_space=pltpu.VMEM),
            pl.BlockSpec(memory_space=pl.ANY),
        ],
        out_specs=pl.BlockSpec(memory_space=pl.ANY),
        scratch_shapes=[
            pltpu.VMEM((m, k), jnp.bfloat16),
            pltpu.VMEM((k, n_local), jnp.bfloat16),
            pltpu.VMEM((2, k, CW), jnp.float32),
            pltpu.VMEM((m, n_local), jnp.bfloat16),
            pltpu.VMEM((m, n_local), jnp.bfloat16),
            pltpu.VMEM((2, ROWS, n_global), jnp.bfloat16),
            pltpu.SemaphoreType.DMA((2,)),
            pltpu.SemaphoreType.DMA((n_wc,)),
            pltpu.SemaphoreType.DMA((n_wc,)),
            pltpu.SemaphoreType.DMA((n_rc - 1,)),
            pltpu.SemaphoreType.DMA((n_rc - 1,)),
            pltpu.SemaphoreType.DMA((2,)),
        ],
        compiler_params=pltpu.CompilerParams(
            collective_id=0,
            vmem_limit_bytes=63 * 1024 * 1024,
        ),
    )(x, W)
